# baseline (speedup 1.0000x reference)
import functools

import jax
import jax.numpy as jnp
from jax import lax
from jax.experimental import pallas as pl
from jax.experimental.pallas import tpu as pltpu
from jax.experimental.pallas import tpu_sc as plsc

F32 = jnp.float32
I32 = jnp.int32

BATCH = 4
NPRED = 4096
NGT = 1024
FH, FW = 222, 124
BINS = FH * FW
NSUB = 16
BINS_PAD = 27648
SLICE = BINS_PAD // NSUB
NVEC = SLICE // 16
PCH = BATCH * NPRED // NSUB // 128
GCH = BATCH * NGT // NSUB // 128


def _frame_idx(yv, zv):
    dy = yv - jnp.min(yv)
    dz = zv - jnp.min(zv)
    iy = jnp.round(dy / jnp.max(dy) * (FH - 1.0)).astype(I32)
    iz = jnp.round(dz / jnp.max(dz) * (FW - 1.0)).astype(I32)
    return iy * FW + iz


def _idx_body(py_ref, pz_ref, gy_ref, gz_ref, idxp_ref, idxg_ref):
    idxp_ref[...] = _frame_idx(py_ref[...], pz_ref[...])
    idxg_ref[...] = _frame_idx(gy_ref[...], gz_ref[...])


_idx_call = pl.pallas_call(
    _idx_body,
    out_shape=[
        jax.ShapeDtypeStruct((NSUB * PCH, 128), I32),
        jax.ShapeDtypeStruct((NSUB * GCH, 128), I32),
    ],
    in_specs=[pl.BlockSpec(memory_space=pltpu.VMEM)] * 4,
)


def _chamfer_body(gt_ref, gt2bf_ref, pbf_ref, b2n_ref, r_ref, cd_ref):
    cd = jnp.float32(0.0)
    for b in range(BATCH):
        a = gt_ref[b]
        ax = a[:, 0:1]
        ay = a[:, 1:2]
        az = a[:, 2:3]
        a2 = ax * ax + ay * ay + az * az
        a2bf = gt2bf_ref[b]
        bbf = pbf_ref[b]
        run_min = None
        bac = jnp.float32(0.0)
        for j in range(NPRED // 1024):
            sl = slice(j * 1024, (j + 1) * 1024)
            b2 = b2n_ref[b:b + 1, sl]
            ab2 = jax.lax.dot_general(
                a2bf, bbf[sl, :], (((1,), (1,)), ((), ())),
                preferred_element_type=F32)
            d = (a2 + b2) - ab2
            mf = jnp.min(d, axis=1, keepdims=True)
            run_min = mf if j == 0 else jnp.minimum(run_min, mf)
            mb = jnp.min(d, axis=0, keepdims=True)
            bac = bac + jnp.sum(jnp.maximum(mb, 0.0))
        cf = jnp.sum(jnp.maximum(run_min, 0.0)) * (1.0 / NGT)
        cb = bac * (1.0 / NPRED)
        cd = cd + (0.8 * cf + 0.2 * cb) / r_ref[0, b]
    cd = cd * (100.0 / BATCH)
    cd_ref[...] = jnp.full((8, 128), cd, F32)


_chamfer_call = pl.pallas_call(
    _chamfer_body,
    out_shape=jax.ShapeDtypeStruct((8, 128), F32),
    in_specs=[pl.BlockSpec(memory_space=pltpu.VMEM)] * 4
    + [pl.BlockSpec(memory_space=pltpu.SMEM)],
)


@functools.cache
def _build_sc_hist():
    return functools.partial(
        pl.kernel,
        mesh=plsc.VectorSubcoreMesh(core_axis_name="c", subcore_axis_name="s"),
        out_type=jax.ShapeDtypeStruct((16,), F32),
        scratch_types=[
            pltpu.VMEM((PCH, 128), I32),
            pltpu.VMEM((GCH, 128), I32),
            pltpu.VMEM((128,), F32),
            pltpu.VMEM((128,), F32),
            pltpu.VMEM((SLICE,), F32),
            pltpu.VMEM((256,), F32),
            pltpu.VMEM((16,), F32),
            pltpu.VMEM_SHARED((BINS_PAD,), F32),
            pltpu.VMEM_SHARED((256,), F32),
        ],
    )(_sc_hist_body)


def _sc_hist_body(idxp_hbm, idxg_hbm, out_hbm,
                  idxp_v, idxg_v, ones_v, negs_v, hist_v, part_v, vec_v,
                  hist_sh, part_sh):
    cid = lax.axis_index("c")
    sid = lax.axis_index("s")

    @pl.when(cid == 0)
    def _():
        zero16 = jnp.zeros((16,), F32)
        one16 = jnp.full((16,), 1.0, F32)
        neg16 = jnp.full((16,), -1.0, F32)
        for i in range(8):
            ones_v[pl.ds(i * 16, 16)] = one16
            negs_v[pl.ds(i * 16, 16)] = neg16
        for i in range(NVEC):
            hist_v[pl.ds(i * 16, 16)] = zero16
        pltpu.sync_copy(hist_v, hist_sh.at[pl.ds(sid * SLICE, SLICE)])
        plsc.subcore_barrier()

        pltpu.sync_copy(idxp_hbm.at[pl.ds(sid * PCH, PCH)], idxp_v)
        pltpu.sync_copy(idxg_hbm.at[pl.ds(sid * GCH, GCH)], idxg_v)
        for j in range(PCH):
            pltpu.sync_copy(ones_v, hist_sh.at[idxp_v.at[j]], add=True)
        for j in range(GCH):
            pltpu.sync_copy(negs_v, hist_sh.at[idxg_v.at[j]], add=True)
        plsc.subcore_barrier()

        pltpu.sync_copy(hist_sh.at[pl.ds(sid * SLICE, SLICE)], hist_v)
        acc = jnp.zeros((16,), F32)
        for i in range(NVEC):
            v = hist_v[pl.ds(i * 16, 16)]
            acc = acc + v * v
        vec_v[...] = acc
        pltpu.sync_copy(vec_v, part_sh.at[pl.ds(sid * 16, 16)])
        plsc.subcore_barrier()

        @pl.when(sid == 0)
        def _():
            pltpu.sync_copy(part_sh, part_v)
            t = jnp.zeros((16,), F32)
            for j in range(16):
                t = t + part_v[pl.ds(j * 16, 16)]
            s = jnp.float32(0.0)
            for j in range(16):
                s = s + t[j]
            vec_v[...] = jnp.full((16,), s * (1.0 / BINS), F32)
            pltpu.sync_copy(vec_v, out_hbm)


def kernel(pred, gt, pcd_radius):
    pred = pred.astype(F32)
    gt = gt.astype(F32)
    px = pred[:, :, 0]
    py = pred[:, :, 1]
    pz = pred[:, :, 2]
    b2n = px * px + py * py + pz * pz
    py2 = py.reshape(NSUB * PCH, 128)
    pz2 = pz.reshape(NSUB * PCH, 128)
    gy2 = gt[:, :, 1].reshape(NSUB * GCH, 128)
    gz2 = gt[:, :, 2].reshape(NSUB * GCH, 128)
    gt2bf = (gt * 2.0).astype(jnp.bfloat16)
    pbf = pred.astype(jnp.bfloat16)
    r = pcd_radius.astype(F32).reshape(1, BATCH)

    idxp, idxg = _idx_call(py2, pz2, gy2, gz2)
    fr16 = _build_sc_hist()(idxp, idxg)
    cd8 = _chamfer_call(gt, gt2bf, pbf, b2n, r)
    return cd8[0, 0] + fr16[0]


# speedup vs baseline: 1.0802x; 1.0802x over previous
import functools

import jax
import jax.numpy as jnp
from jax import lax
from jax.experimental import pallas as pl
from jax.experimental.pallas import tpu as pltpu
from jax.experimental.pallas import tpu_sc as plsc

F32 = jnp.float32
I32 = jnp.int32

BATCH = 4
NPRED = 4096
NGT = 1024
FH, FW = 222, 124
BINS = FH * FW
NSUB = 16
BINS_PAD = 27648
SLICE = BINS_PAD // NSUB
NVEC = SLICE // 16
PCH = BATCH * NPRED // NSUB // 128
GCH = BATCH * NGT // NSUB // 128


def _frame_idx(yv, zv):
    dy = yv - jnp.min(yv)
    dz = zv - jnp.min(zv)
    iy = jnp.round(dy / jnp.max(dy) * (FH - 1.0)).astype(I32)
    iz = jnp.round(dz / jnp.max(dz) * (FW - 1.0)).astype(I32)
    return iy * FW + iz


def _idx_body(py_ref, pz_ref, gy_ref, gz_ref, idxp_ref, idxg_ref):
    idxp_ref[...] = _frame_idx(py_ref[...], pz_ref[...])
    idxg_ref[...] = _frame_idx(gy_ref[...], gz_ref[...])


_idx_call = pl.pallas_call(
    _idx_body,
    out_shape=[
        jax.ShapeDtypeStruct((NSUB * PCH, 128), I32),
        jax.ShapeDtypeStruct((NSUB * GCH, 128), I32),
    ],
    in_specs=[pl.BlockSpec(memory_space=pltpu.VMEM)] * 4,
)


def _chamfer_body(gt_ref, gt2bf_ref, ptbf_ref, px_ref, py_ref, pz_ref,
                  r_ref, cd_ref):
    cd = jnp.float32(0.0)
    for b in range(BATCH):
        a = gt_ref[b]
        ax = a[:, 0:1]
        ay = a[:, 1:2]
        az = a[:, 2:3]
        a2 = ax * ax + ay * ay + az * az
        a2bf = gt2bf_ref[b]
        bT = ptbf_ref[b]
        run_min = None
        bac = jnp.float32(0.0)
        for j in range(NPRED // 1024):
            sl = slice(j * 1024, (j + 1) * 1024)
            bx = px_ref[b:b + 1, sl]
            by = py_ref[b:b + 1, sl]
            bz = pz_ref[b:b + 1, sl]
            b2 = bx * bx + by * by + bz * bz
            ab2 = jax.lax.dot_general(
                a2bf, bT[:, sl], (((1,), (0,)), ((), ())),
                preferred_element_type=F32)
            d = (a2 + b2) - ab2
            mf = jnp.min(d, axis=1, keepdims=True)
            run_min = mf if j == 0 else jnp.minimum(run_min, mf)
            mb = jnp.min(d, axis=0, keepdims=True)
            bac = bac + jnp.sum(jnp.maximum(mb, 0.0))
        cf = jnp.sum(jnp.maximum(run_min, 0.0)) * (1.0 / NGT)
        cb = bac * (1.0 / NPRED)
        cd = cd + (0.8 * cf + 0.2 * cb) / r_ref[0, b]
    cd = cd * (100.0 / BATCH)
    cd_ref[...] = jnp.full((8, 128), cd, F32)


_chamfer_call = pl.pallas_call(
    _chamfer_body,
    out_shape=jax.ShapeDtypeStruct((8, 128), F32),
    in_specs=[pl.BlockSpec(memory_space=pltpu.VMEM)] * 6
    + [pl.BlockSpec(memory_space=pltpu.SMEM)],
)


@functools.cache
def _build_sc_hist():
    return functools.partial(
        pl.kernel,
        mesh=plsc.VectorSubcoreMesh(core_axis_name="c", subcore_axis_name="s"),
        out_type=jax.ShapeDtypeStruct((16,), F32),
        scratch_types=[
            pltpu.VMEM((PCH, 128), I32),
            pltpu.VMEM((GCH, 128), I32),
            pltpu.VMEM((128,), F32),
            pltpu.VMEM((128,), F32),
            pltpu.VMEM((SLICE,), F32),
            pltpu.VMEM((256,), F32),
            pltpu.VMEM((16,), F32),
            pltpu.VMEM_SHARED((BINS_PAD,), F32),
            pltpu.VMEM_SHARED((256,), F32),
        ],
    )(_sc_hist_body)


def _sc_hist_body(idxp_hbm, idxg_hbm, out_hbm,
                  idxp_v, idxg_v, ones_v, negs_v, hist_v, part_v, vec_v,
                  hist_sh, part_sh):
    cid = lax.axis_index("c")
    sid = lax.axis_index("s")

    @pl.when(cid == 0)
    def _():
        zero16 = jnp.zeros((16,), F32)
        one16 = jnp.full((16,), 1.0, F32)
        neg16 = jnp.full((16,), -1.0, F32)
        for i in range(8):
            ones_v[pl.ds(i * 16, 16)] = one16
            negs_v[pl.ds(i * 16, 16)] = neg16
        for i in range(NVEC):
            hist_v[pl.ds(i * 16, 16)] = zero16
        pltpu.sync_copy(hist_v, hist_sh.at[pl.ds(sid * SLICE, SLICE)])
        plsc.subcore_barrier()

        pltpu.sync_copy(idxp_hbm.at[pl.ds(sid * PCH, PCH)], idxp_v)
        pltpu.sync_copy(idxg_hbm.at[pl.ds(sid * GCH, GCH)], idxg_v)
        for j in range(PCH):
            pltpu.sync_copy(ones_v, hist_sh.at[idxp_v.at[j]], add=True)
        for j in range(GCH):
            pltpu.sync_copy(negs_v, hist_sh.at[idxg_v.at[j]], add=True)
        plsc.subcore_barrier()

        pltpu.sync_copy(hist_sh.at[pl.ds(sid * SLICE, SLICE)], hist_v)
        acc = jnp.zeros((16,), F32)
        for i in range(NVEC):
            v = hist_v[pl.ds(i * 16, 16)]
            acc = acc + v * v
        vec_v[...] = acc
        pltpu.sync_copy(vec_v, part_sh.at[pl.ds(sid * 16, 16)])
        plsc.subcore_barrier()

        @pl.when(sid == 0)
        def _():
            pltpu.sync_copy(part_sh, part_v)
            t = jnp.zeros((16,), F32)
            for j in range(16):
                t = t + part_v[pl.ds(j * 16, 16)]
            s = jnp.float32(0.0)
            for j in range(16):
                s = s + t[j]
            vec_v[...] = jnp.full((16,), s * (1.0 / BINS), F32)
            pltpu.sync_copy(vec_v, out_hbm)


def kernel(pred, gt, pcd_radius):
    pred = pred.astype(F32)
    gt = gt.astype(F32)
    px = pred[:, :, 0]
    py = pred[:, :, 1]
    pz = pred[:, :, 2]
    py2 = py.reshape(NSUB * PCH, 128)
    pz2 = pz.reshape(NSUB * PCH, 128)
    gy2 = gt[:, :, 1].reshape(NSUB * GCH, 128)
    gz2 = gt[:, :, 2].reshape(NSUB * GCH, 128)
    gt2bf = (gt * 2.0).astype(jnp.bfloat16)
    ptbf = jnp.transpose(pred, (0, 2, 1)).astype(jnp.bfloat16)
    r = pcd_radius.astype(F32).reshape(1, BATCH)

    idxp, idxg = _idx_call(py2, pz2, gy2, gz2)
    fr16 = _build_sc_hist()(idxp, idxg)
    cd8 = _chamfer_call(gt, gt2bf, ptbf, px, py, pz, r)
    return cd8[0, 0] + fr16[0]
